# baseline (speedup 1.0000x reference)
import functools

import jax
import jax.numpy as jnp
from jax import lax
from jax.experimental import pallas as pl
from jax.experimental.pallas import tpu as pltpu
from jax.experimental.pallas import tpu_sc as plsc

_J, _F, _C, _B = 25, 32768, 64, 16
_NC, _NS = 2, 16
_NW = _NC * _NS
_FPW = _F // _NW
_BF = 128
_NBLK = _FPW // _BF
_ROWS = _J * _B
_PR = 200
_CROW = _PR
_PRPAD = 208
_NG = _C // 16
_POLL_CAP = 1500


def _mesh():
    return plsc.VectorSubcoreMesh(
        core_axis_name="c", subcore_axis_name="s",
        num_cores=_NC, num_subcores=_NS)


def _zero_rows(ref, nrows):
    z = jnp.zeros((16,), jnp.float32)

    def body(r, _):
        for g in range(8):
            ref[r, pl.ds(g * 16, 16)] = z
        return 0

    lax.fori_loop(0, nrows, body, 0)


def _tree_reduce_rows(shared, tmp, racc, nrows, rlo):
    _zero_rows(racc, nrows)
    for t in range(_NS):
        pltpu.sync_copy(shared.at[t, pl.ds(rlo, nrows)],
                        tmp.at[pl.ds(0, nrows)])

        def rbody(r, _):
            for g in range(8):
                racc[r, pl.ds(g * 16, 16)] = (
                    racc[r, pl.ds(g * 16, 16)] + tmp[r, pl.ds(g * 16, 16)])
            return 0

        lax.fori_loop(0, nrows, rbody, 0)


def _count_and_bounds(idxv, cnt, bound):
    for b in range(_B):
        cnt[b] = 0

    def cbody(gi, _):
        segv = idxv[pl.ds(gi * 16, 16)]
        for l in range(16):
            seg = segv[l]
            cnt[seg] = cnt[seg] + 1
        return 0

    lax.fori_loop(0, _FPW // 16, cbody, 0)
    bound[0] = 0
    for b in range(_B):
        bound[b + 1] = bound[b] + cnt[b]


def _pipelined_task_loop(x_hbm, f0, buf0, buf1, sem0, sem1, process):
    ntask = _J * _NBLK

    def copy(task, buf, sem):
        j = task // _NBLK
        blk = lax.rem(task, _NBLK)
        return pltpu.async_copy(
            x_hbm.at[j, pl.ds(f0 + blk * _BF, _BF)], buf, sem)

    copy(jnp.int32(0), buf0, sem0)

    def body(it, _):
        a = it * 2
        copy(a + 1, buf1, sem1)
        pltpu.make_async_copy(x_hbm.at[0, pl.ds(f0, _BF)], buf0, sem0).wait()
        process(a, buf0)

        @pl.when(a + 2 < ntask)
        def _():
            copy(a + 2, buf0, sem0)

        pltpu.make_async_copy(x_hbm.at[0, pl.ds(f0, _BF)], buf1, sem1).wait()
        process(a + 1, buf1)
        return 0

    lax.fori_loop(0, ntask // 2, body, 0)


def _block_seg_range(idxv, ib):
    seg_first = idxv[pl.ds(ib, 16)][0]
    seg_last = idxv[pl.ds(ib + (_BF - 16), 16)][15]
    return seg_first, seg_last


def _lane_splat(v, lane):
    idx = jnp.full((16,), lane, jnp.int32)
    return lax.gather(
        v, idx[:, None],
        lax.GatherDimensionNumbers(
            offset_dims=(), collapsed_slice_dims=(0,), start_index_map=(0,)),
        slice_sizes=(1,), mode=lax.GatherScatterMode.PROMISE_IN_BOUNDS)


def _sc_fused(x, batch_index, w_packed):
    @functools.partial(
        pl.kernel,
        out_type=(
            jax.ShapeDtypeStruct((_NC, _PRPAD, 2 * _C), jnp.float32),
            jax.ShapeDtypeStruct((_NC, _PR, 2 * _C), jnp.float32),
            jax.ShapeDtypeStruct((_NC, 16), jnp.int32),
        ),
        mesh=_mesh(),
        scratch_types=[
            pltpu.VMEM((_BF, _C), jnp.float32),
            pltpu.VMEM((_BF, _C), jnp.float32),
            pltpu.VMEM((_PRPAD, 2 * _C), jnp.float32),
            pltpu.VMEM((_FPW,), jnp.int32),
            pltpu.VMEM((16, 2 * _C), jnp.float32),
            pltpu.VMEM((16, 2 * _C), jnp.float32),
            pltpu.VMEM((32, 2 * _C), jnp.float32),
            pltpu.VMEM((16, 2 * _C), jnp.float32),
            pltpu.VMEM((16,), jnp.int32),
            pltpu.VMEM_SHARED((_NS, _PRPAD, 2 * _C), jnp.float32),
            pltpu.VMEM_SHARED((_PRPAD, 2 * _C), jnp.float32),
            pltpu.SMEM((_B,), jnp.int32),
            pltpu.SMEM((_B + 1,), jnp.int32),
            pltpu.SemaphoreType.DMA,
            pltpu.SemaphoreType.DMA,
        ],
    )
    def ker(x_hbm, idx_hbm, w_hbm, pa, pb, flag, buf0, buf1, acc, idxv,
            tmp, racc, w2v, aux, fbuf, shared, sgc, cnt, bound, sem0, sem1):
        cid = lax.axis_index("c")
        sid = lax.axis_index("s")
        wid = sid * _NC + cid
        f0 = wid * _FPW

        @pl.when(sid == 15)
        def _():
            fbuf[pl.ds(0, 16)] = jnp.zeros((16,), jnp.int32)
            pltpu.sync_copy(fbuf, flag.at[cid])

        _zero_rows(acc, _PRPAD)
        pltpu.sync_copy(idx_hbm.at[pl.ds(f0, _FPW)], idxv)
        pltpu.sync_copy(w_hbm, w2v)
        _count_and_bounds(idxv, cnt, bound)

        for b in range(_B):
            v = jnp.full((16,), cnt[b].astype(jnp.float32))
            for g in range(_NG):
                acc[_CROW + b // 2, pl.ds((b % 2) * _C + g * 16, 16)] = v

        zv = jnp.zeros((16,), jnp.float32)

        def process1(task, buf):
            j = task // _NBLK
            blk = lax.rem(task, _NBLK)
            jpr = j * (_B // 2)
            ib = blk * _BF
            seg_first, seg_last = _block_seg_range(idxv, ib)

            def segbody(seg, _):
                lo = jnp.maximum(bound[seg], ib) - ib
                hi = jnp.minimum(bound[seg + 1], ib + _BF) - ib
                pr = jpr + (seg >> 1)
                lb = (seg & 1) * _C

                @plsc.parallel_loop(lo, hi, unroll=4, carry=(zv,) * _NG)
                def c(f, c):
                    return tuple(
                        c[g] + buf[f, pl.ds(g * 16, 16)] for g in range(_NG))

                for g in range(_NG):
                    acc[pr, pl.ds(lb + g * 16, 16)] = (
                        acc[pr, pl.ds(lb + g * 16, 16)] + c[g])
                return 0

            lax.fori_loop(seg_first, seg_last + 1, segbody, 0)

        _pipelined_task_loop(x_hbm, f0, buf0, buf1, sem0, sem1, process1)

        pltpu.sync_copy(acc, shared.at[sid])
        plsc.subcore_barrier()

        @pl.when(sid < 13)
        def _():
            rlo = sid * 16
            _tree_reduce_rows(shared, tmp, racc, 16, rlo)
            pltpu.sync_copy(racc, pa.at[cid, pl.ds(rlo, 16)])

        plsc.subcore_barrier()

        @pl.when(sid == 15)
        def _():
            fbuf[pl.ds(0, 16)] = jnp.ones((16,), jnp.int32)
            pltpu.sync_copy(fbuf, flag.at[cid])

        def pbody(i, done):
            @pl.when(done == 0)
            def _():
                pltpu.sync_copy(flag.at[1 - cid], fbuf)

            return fbuf[pl.ds(0, 16)][0]

        lax.fori_loop(0, _POLL_CAP, pbody, jnp.int32(0))

        pltpu.sync_copy(pa.at[0, pl.ds(_CROW, 8)], tmp.at[pl.ds(0, 8)])
        pltpu.sync_copy(pa.at[1, pl.ds(_CROW, 8)], tmp.at[pl.ds(8, 8)])
        for r in range(8):
            for g in range(8):
                aux[r, pl.ds(g * 16, 16)] = jnp.maximum(
                    tmp[r, pl.ds(g * 16, 16)] + tmp[8 + r, pl.ds(g * 16, 16)],
                    1.0)

        @pl.when(sid < 13)
        def _():
            rlo = sid * 16
            pltpu.sync_copy(pa.at[0, pl.ds(rlo, 16)], tmp)
            pltpu.sync_copy(pa.at[1, pl.ds(rlo, 16)], racc)
            for r in range(16):
                for g in range(8):
                    tmp[r, pl.ds(g * 16, 16)] = (
                        (tmp[r, pl.ds(g * 16, 16)]
                         + racc[r, pl.ds(g * 16, 16)])
                        / aux[r % 8, pl.ds(g * 16, 16)])
            zv16 = jnp.zeros((16,), jnp.float32)
            for r in range(16):

                @plsc.parallel_loop(0, 32, carry=(zv16,) * 8)
                def cg(q, c):
                    k0 = 2 * q
                    grp = (q >> 3) * 16
                    lane = k0 & 15
                    new = []
                    for h in range(2):
                        mg = tmp[r, pl.ds(h * _C + grp, 16)]
                        s0 = _lane_splat(mg, lane)
                        s1 = _lane_splat(mg, lane + 1)
                        for g in range(_NG):
                            w0 = w2v[q, pl.ds(g * 16, 16)]
                            w1 = w2v[q, pl.ds(_C + g * 16, 16)]
                            new.append(c[h * _NG + g] + s0 * w0 + s1 * w1)
                    return tuple(new)

                for i in range(8):
                    e = jnp.exp(2.0 * cg[i])
                    racc[r, pl.ds(i * 16, 16)] = 1.0 - 2.0 / (e + 1.0)

            @pl.when(sid < 12)
            def _():
                pltpu.sync_copy(racc, sgc.at[pl.ds(rlo, 16)])

            @pl.when(sid == 12)
            def _():
                pltpu.sync_copy(racc.at[pl.ds(0, 8)], sgc.at[pl.ds(192, 8)])

        plsc.subcore_barrier()

        _zero_rows(acc, _PRPAD)
        perms = [lax.iota(jnp.int32, 16) ^ sh for sh in (8, 4, 2, 1)]
        gdims = lax.GatherDimensionNumbers(
            offset_dims=(), collapsed_slice_dims=(0,), start_index_map=(0,))

        def process2(task, buf):
            j = task // _NBLK
            blk = lax.rem(task, _NBLK)
            jpr = j * (_B // 2)
            ib = blk * _BF

            @pl.when(blk == 0)
            def _():
                pltpu.sync_copy(sgc.at[pl.ds(jpr, 16)], aux)

            seg_first, seg_last = _block_seg_range(idxv, ib)

            def segbody(seg, _):
                lo = jnp.maximum(bound[seg], ib) - ib
                hi = jnp.minimum(bound[seg + 1], ib + _BF) - ib
                pr = jpr + (seg >> 1)
                lb = (seg & 1) * _C
                gg = [aux[seg >> 1, pl.ds(lb + g * 16, 16)]
                      for g in range(_NG)]

                @plsc.parallel_loop(lo, hi, unroll=4, carry=(zv,) * _NG)
                def c(f, c):
                    xg = [buf[f, pl.ds(g * 16, 16)] for g in range(_NG)]
                    prod = xg[0] * gg[0]
                    for g in range(1, _NG):
                        prod = prod + xg[g] * gg[g]
                    for perm in perms:
                        prod = prod + lax.gather(
                            prod, perm[:, None], gdims, slice_sizes=(1,),
                            mode=lax.GatherScatterMode.PROMISE_IN_BOUNDS)
                    gate = 1.0 / (1.0 + jnp.exp(-prod))
                    return tuple(c[g] + gate * xg[g] for g in range(_NG))

                for g in range(_NG):
                    acc[pr, pl.ds(lb + g * 16, 16)] = (
                        acc[pr, pl.ds(lb + g * 16, 16)] + c[g])
                return 0

            lax.fori_loop(seg_first, seg_last + 1, segbody, 0)

        _pipelined_task_loop(x_hbm, f0, buf0, buf1, sem0, sem1, process2)

        pltpu.sync_copy(acc, shared.at[sid])
        plsc.subcore_barrier()

        @pl.when(sid < 12)
        def _():
            rlo = sid * 16
            _tree_reduce_rows(shared, tmp, racc, 16, rlo)
            pltpu.sync_copy(racc, pb.at[cid, pl.ds(rlo, 16)])

        @pl.when(sid == 12)
        def _():
            _tree_reduce_rows(shared, tmp, racc, 8, 192)
            pltpu.sync_copy(racc.at[pl.ds(0, 8)], pb.at[cid, pl.ds(192, 8)])

    return ker(x, batch_index, w_packed)


def _tc_fin(part_b, part_a):
    def ker(pb_ref, pa_ref, out_ref):
        p = pb_ref[0] + pb_ref[1]
        pa = pa_ref[0] + pa_ref[1]
        cntm = jnp.maximum(pa[_CROW:_CROW + _B // 2, :], 1.0)
        dvs = jnp.concatenate([cntm] * _J, axis=0)
        out_ref[...] = p / dvs

    return pl.pallas_call(
        ker,
        out_shape=jax.ShapeDtypeStruct((_PR, 2 * _C), jnp.float32),
    )(part_b, part_a)


def kernel(x, batch_index, W):
    idx = batch_index.astype(jnp.int32)
    w2 = W.reshape(32, 2 * _C)
    part_a, part_b, _ = _sc_fused(x, idx, w2)
    out = _tc_fin(part_b, part_a)
    return out.reshape(_J, _B, _C)


# speedup vs baseline: 1.1267x; 1.1267x over previous
import functools

import jax
import jax.numpy as jnp
from jax import lax
from jax.experimental import pallas as pl
from jax.experimental.pallas import tpu as pltpu
from jax.experimental.pallas import tpu_sc as plsc

_J, _F, _C, _B = 25, 32768, 64, 16
_NC, _NS = 2, 16
_NW = _NC * _NS
_FPW = _F // _NW
_ROWS = _J * _B
_PR = 200
_CROW = _PR
_PRPAD = 208
_PRPS = 16
_NRT = _PRPAD // _PRPS
_NG = _C // 16


def _mesh():
    return plsc.VectorSubcoreMesh(
        core_axis_name="c", subcore_axis_name="s",
        num_cores=_NC, num_subcores=_NS)


def _zero_rows(ref, nrows):
    z = jnp.zeros((16,), jnp.float32)

    def body(r, _):
        for g in range(8):
            ref[r, pl.ds(g * 16, 16)] = z
        return 0

    lax.fori_loop(0, nrows, body, 0)


def _stage_reduce_store(acc, shared, tmp, racc, out, cid, sid):
    pltpu.sync_copy(acc, shared.at[sid])
    plsc.subcore_barrier()

    @pl.when(sid < _NRT)
    def _():
        rlo = sid * _PRPS
        _zero_rows(racc, _PRPS)
        for t in range(_NS):
            pltpu.sync_copy(shared.at[t, pl.ds(rlo, _PRPS)], tmp)

            def rbody(r, _):
                for g in range(8):
                    racc[r, pl.ds(g * 16, 16)] = (
                        racc[r, pl.ds(g * 16, 16)] + tmp[r, pl.ds(g * 16, 16)])
                return 0

            lax.fori_loop(0, _PRPS, rbody, 0)
        pltpu.sync_copy(racc, out.at[cid, pl.ds(rlo, _PRPS)])


def _count_and_bounds(idxv, cnt, bound):
    for b in range(_B):
        cnt[b] = 0

    def cbody(gi, _):
        segv = idxv[pl.ds(gi * 16, 16)]
        for l in range(16):
            seg = segv[l]
            cnt[seg] = cnt[seg] + 1
        return 0

    lax.fori_loop(0, _FPW // 16, cbody, 0)
    bound[0] = 0
    for b in range(_B):
        bound[b + 1] = bound[b] + cnt[b]


def _pipelined_task_loop(x_hbm, f0, bf, buf0, buf1, sem0, sem1, process):
    nblk = _FPW // bf
    ntask = _J * nblk

    def copy(task, buf, sem):
        j = task // nblk
        blk = lax.rem(task, nblk)
        return pltpu.async_copy(
            x_hbm.at[j, pl.ds(f0 + blk * bf, bf)], buf, sem)

    copy(jnp.int32(0), buf0, sem0)

    def body(it, _):
        a = it * 2
        copy(a + 1, buf1, sem1)
        pltpu.make_async_copy(x_hbm.at[0, pl.ds(f0, bf)], buf0, sem0).wait()
        process(a, buf0)

        @pl.when(a + 2 < ntask)
        def _():
            copy(a + 2, buf0, sem0)

        pltpu.make_async_copy(x_hbm.at[0, pl.ds(f0, bf)], buf1, sem1).wait()
        process(a + 1, buf1)
        return 0

    lax.fori_loop(0, ntask // 2, body, 0)


def _block_seg_range(idxv, ib, bf):
    seg_first = idxv[pl.ds(ib, 16)][0]
    seg_last = idxv[pl.ds(ib + (bf - 16), 16)][15]
    return seg_first, seg_last


def _sc_pass1(x, batch_index):
    bf = 256
    nblk = _FPW // bf

    @functools.partial(
        pl.kernel,
        out_type=jax.ShapeDtypeStruct((_NC, _PRPAD, 2 * _C), jnp.float32),
        mesh=_mesh(),
        scratch_types=[
            pltpu.VMEM((bf, _C), jnp.float32),
            pltpu.VMEM((bf, _C), jnp.float32),
            pltpu.VMEM((_PRPAD, 2 * _C), jnp.float32),
            pltpu.VMEM((_FPW,), jnp.int32),
            pltpu.VMEM((_PRPS, 2 * _C), jnp.float32),
            pltpu.VMEM((_PRPS, 2 * _C), jnp.float32),
            pltpu.VMEM_SHARED((_NS, _PRPAD, 2 * _C), jnp.float32),
            pltpu.SMEM((_B,), jnp.int32),
            pltpu.SMEM((_B + 1,), jnp.int32),
            pltpu.SemaphoreType.DMA,
            pltpu.SemaphoreType.DMA,
        ],
    )
    def ker(x_hbm, idx_hbm, out, buf0, buf1, acc, idxv, tmp, racc,
            shared, cnt, bound, sem0, sem1):
        cid = lax.axis_index("c")
        sid = lax.axis_index("s")
        wid = sid * _NC + cid
        f0 = wid * _FPW

        _zero_rows(acc, _PRPAD)
        pltpu.sync_copy(idx_hbm.at[pl.ds(f0, _FPW)], idxv)
        _count_and_bounds(idxv, cnt, bound)

        for b in range(_B):
            v = jnp.full((16,), cnt[b].astype(jnp.float32))
            for g in range(_NG):
                acc[_CROW + b // 2, pl.ds((b % 2) * _C + g * 16, 16)] = v

        zv = jnp.zeros((16,), jnp.float32)

        def process(task, buf):
            j = task // nblk
            blk = lax.rem(task, nblk)
            jpr = j * (_B // 2)
            ib = blk * bf
            seg_first, seg_last = _block_seg_range(idxv, ib, bf)

            def segbody(seg, _):
                lo = jnp.maximum(bound[seg], ib) - ib
                hi = jnp.minimum(bound[seg + 1], ib + bf) - ib
                pr = jpr + (seg >> 1)
                lb = (seg & 1) * _C

                @plsc.parallel_loop(lo, hi, unroll=4, carry=(zv,) * _NG)
                def c(f, c):
                    return tuple(
                        c[g] + buf[f, pl.ds(g * 16, 16)] for g in range(_NG))
                for g in range(_NG):
                    acc[pr, pl.ds(lb + g * 16, 16)] = (
                        acc[pr, pl.ds(lb + g * 16, 16)] + c[g])
                return 0

            lax.fori_loop(seg_first, seg_last + 1, segbody, 0)

        _pipelined_task_loop(x_hbm, f0, bf, buf0, buf1, sem0, sem1, process)
        _stage_reduce_store(acc, shared, tmp, racc, out, cid, sid)

    return ker(x, batch_index)


def _sc_pass2(x, batch_index, gc_packed):
    bf = 256
    nblk = _FPW // bf

    @functools.partial(
        pl.kernel,
        out_type=jax.ShapeDtypeStruct((_NC, _PRPAD, 2 * _C), jnp.float32),
        mesh=_mesh(),
        scratch_types=[
            pltpu.VMEM((bf, _C), jnp.float32),
            pltpu.VMEM((bf, _C), jnp.float32),
            pltpu.VMEM((_PRPAD, 2 * _C), jnp.float32),
            pltpu.VMEM((16, 2 * _C), jnp.float32),
            pltpu.VMEM((_FPW,), jnp.int32),
            pltpu.VMEM((_PRPS, 2 * _C), jnp.float32),
            pltpu.VMEM((_PRPS, 2 * _C), jnp.float32),
            pltpu.VMEM_SHARED((_NS, _PRPAD, 2 * _C), jnp.float32),
            pltpu.SMEM((_B,), jnp.int32),
            pltpu.SMEM((_B + 1,), jnp.int32),
            pltpu.SemaphoreType.DMA,
            pltpu.SemaphoreType.DMA,
        ],
    )
    def ker(x_hbm, idx_hbm, gc_hbm, out, buf0, buf1, acc, aux, idxv,
            tmp, racc, shared, cnt, bound, sem0, sem1):
        cid = lax.axis_index("c")
        sid = lax.axis_index("s")
        wid = sid * _NC + cid
        f0 = wid * _FPW

        _zero_rows(acc, _PRPAD)
        pltpu.sync_copy(idx_hbm.at[pl.ds(f0, _FPW)], idxv)
        _count_and_bounds(idxv, cnt, bound)

        zv = jnp.zeros((16,), jnp.float32)
        perms = [lax.iota(jnp.int32, 16) ^ sh for sh in (8, 4, 2, 1)]
        gdims = lax.GatherDimensionNumbers(
            offset_dims=(), collapsed_slice_dims=(0,), start_index_map=(0,))

        def process(task, buf):
            j = task // nblk
            blk = lax.rem(task, nblk)
            jpr = j * (_B // 2)
            ib = blk * bf

            @pl.when(blk == 0)
            def _():
                pltpu.sync_copy(gc_hbm.at[pl.ds(jpr, 16)], aux)

            seg_first, seg_last = _block_seg_range(idxv, ib, bf)

            def segbody(seg, _):
                lo = jnp.maximum(bound[seg], ib) - ib
                hi = jnp.minimum(bound[seg + 1], ib + bf) - ib
                pr = jpr + (seg >> 1)
                lb = (seg & 1) * _C
                gg = [aux[seg >> 1, pl.ds(lb + g * 16, 16)]
                      for g in range(_NG)]

                @plsc.parallel_loop(lo, hi, unroll=4, carry=(zv,) * _NG)
                def c(f, c):
                    xg = [buf[f, pl.ds(g * 16, 16)] for g in range(_NG)]
                    prod = xg[0] * gg[0]
                    for g in range(1, _NG):
                        prod = prod + xg[g] * gg[g]
                    for perm in perms:
                        prod = prod + lax.gather(
                            prod, perm[:, None], gdims, slice_sizes=(1,),
                            mode=lax.GatherScatterMode.PROMISE_IN_BOUNDS)
                    gate = 1.0 / (1.0 + jnp.exp(-prod))
                    return tuple(c[g] + gate * xg[g] for g in range(_NG))
                for g in range(_NG):
                    acc[pr, pl.ds(lb + g * 16, 16)] = (
                        acc[pr, pl.ds(lb + g * 16, 16)] + c[g])
                return 0

            lax.fori_loop(seg_first, seg_last + 1, segbody, 0)

        _pipelined_task_loop(x_hbm, f0, bf, buf0, buf1, sem0, sem1, process)
        _stage_reduce_store(acc, shared, tmp, racc, out, cid, sid)

    return ker(x, batch_index, gc_packed)


def _divisors(pa):
    cntm = jnp.maximum(pa[_CROW:_CROW + _B // 2, :], 1.0)
    return jnp.concatenate([cntm] * _J, axis=0)


def _tc_mid(part_a, w):
    def ker(pa_ref, w_ref, gc_ref):
        p = pa_ref[0] + pa_ref[1]
        cntm = jnp.maximum(p[_CROW:_CROW + _B // 2, :], 1.0)
        dvs = jnp.concatenate([cntm] * (_J + 1), axis=0)
        means = p / dvs
        wv = w_ref[...]
        z = jnp.zeros((_C, _C), jnp.float32)
        w2 = jnp.concatenate(
            [jnp.concatenate([wv, z], axis=1),
             jnp.concatenate([z, wv], axis=1)], axis=0)
        gc_ref[...] = jnp.tanh(
            jnp.dot(means, w2, preferred_element_type=jnp.float32))

    return pl.pallas_call(
        ker,
        out_shape=jax.ShapeDtypeStruct((_PRPAD, 2 * _C), jnp.float32),
    )(part_a, w)


def _tc_fin(part_b, part_a):
    def ker(pb_ref, pa_ref, out_ref):
        p = pb_ref[0] + pb_ref[1]
        pa = pa_ref[0] + pa_ref[1]
        out_ref[...] = p[:_PR, :] / _divisors(pa)

    return pl.pallas_call(
        ker,
        out_shape=jax.ShapeDtypeStruct((_PR, 2 * _C), jnp.float32),
    )(part_b, part_a)


def kernel(x, batch_index, W):
    idx = batch_index.astype(jnp.int32)
    part_a = _sc_pass1(x, idx)
    gc = _tc_mid(part_a, W)
    part_b = _sc_pass2(x, idx, gc)
    out = _tc_fin(part_b, part_a)
    return out.reshape(_J, _B, _C)
